# baseline (speedup 1.0000x reference)
import functools
import math

import jax
import jax.numpy as jnp
from jax import lax
from jax.experimental import pallas as pl
from jax.experimental.pallas import tpu as pltpu
from jax.experimental.pallas import tpu_sc as plsc

_B, _T, _C = 1, 2048, 1024
_H, _KVH, _D = 16, 4, 64
_E, _K, _F = 8, 2, 2048
_EPS = 1e-5
_BT = 256
_BQ, _BK = 256, 256
_NQ, _NK = _T // _BQ, _T // _BK
_BS = 256
_NB = (_K * _T) // _BS + _E
_P = _NB * _BS
_NEG = -1e30


def _proj_body(x_ref, w_ref, wq_ref, wk_ref, wv_ref, cq_ref, sq_ref,
               ck_ref, sk_ref, q_ref, k_ref, v_ref):
    x = x_ref[...]
    nrm = jnp.mean(x * x, axis=-1, keepdims=True)
    h = (x * lax.rsqrt(nrm + _EPS) * w_ref[...]).astype(jnp.bfloat16)
    q = jnp.dot(h, wq_ref[...], preferred_element_type=jnp.float32)
    k = jnp.dot(h, wk_ref[...], preferred_element_type=jnp.float32)
    v = jnp.dot(h, wv_ref[...], preferred_element_type=jnp.float32)
    hq = _H * _D // 2
    hk = _KVH * _D // 2
    qe, qo = q[:, :hq], q[:, hq:]
    ke, ko = k[:, :hk], k[:, hk:]
    cq, sq = cq_ref[...], sq_ref[...]
    ck, sk = ck_ref[...], sk_ref[...]
    q_ref[:, :hq] = qe * cq - qo * sq
    q_ref[:, hq:] = qe * sq + qo * cq
    k_ref[:, :hk] = ke * ck - ko * sk
    k_ref[:, hk:] = ke * sk + ko * ck
    v_ref[...] = v


def _proj(x2d, ln1_w, wq_p, wk_p, wv, cq, sq, ck, sk):
    n = _T // _BT
    return pl.pallas_call(
        _proj_body,
        grid=(n,),
        in_specs=[
            pl.BlockSpec((_BT, _C), lambda i: (i, 0)),
            pl.BlockSpec((1, _C), lambda i: (0, 0)),
            pl.BlockSpec((_C, _H * _D), lambda i: (0, 0)),
            pl.BlockSpec((_C, _KVH * _D), lambda i: (0, 0)),
            pl.BlockSpec((_C, _KVH * _D), lambda i: (0, 0)),
            pl.BlockSpec((_BT, _H * _D // 2), lambda i: (i, 0)),
            pl.BlockSpec((_BT, _H * _D // 2), lambda i: (i, 0)),
            pl.BlockSpec((_BT, _KVH * _D // 2), lambda i: (i, 0)),
            pl.BlockSpec((_BT, _KVH * _D // 2), lambda i: (i, 0)),
        ],
        out_specs=[
            pl.BlockSpec((_BT, _H * _D), lambda i: (i, 0)),
            pl.BlockSpec((_BT, _KVH * _D), lambda i: (i, 0)),
            pl.BlockSpec((_BT, _KVH * _D), lambda i: (i, 0)),
        ],
        out_shape=[
            jax.ShapeDtypeStruct((_T, _H * _D), jnp.float32),
            jax.ShapeDtypeStruct((_T, _KVH * _D), jnp.float32),
            jax.ShapeDtypeStruct((_T, _KVH * _D), jnp.float32),
        ],
    )(x2d, ln1_w, wq_p, wk_p, wv, cq, sq, ck, sk)


_REP = _H // _KVH
_RQ = _REP * _BQ
_BK2 = 512
_NK2 = _T // _BK2


def _attn_body(q_ref, k_ref, v_ref, o_ref, m_ref, l_ref, acc_ref):
    qi = pl.program_id(1)
    kb = pl.program_id(2)
    hi = qi // (_BK2 // _BQ)

    @pl.when(kb == 0)
    def _init():
        m_ref[...] = jnp.full_like(m_ref, _NEG)
        l_ref[...] = jnp.zeros_like(l_ref)
        acc_ref[...] = jnp.zeros_like(acc_ref)

    def _update(s):
        v = v_ref[0].astype(jnp.bfloat16)
        m_prev = m_ref[...]
        m_new = jnp.maximum(m_prev, jnp.max(s, axis=1, keepdims=True))
        alpha = jnp.exp(m_prev - m_new)
        p = jnp.exp(s - m_new)
        l_ref[...] = l_ref[...] * alpha + jnp.sum(p, axis=1, keepdims=True)
        acc_ref[...] = acc_ref[...] * alpha + jnp.dot(
            p.astype(jnp.bfloat16), v, preferred_element_type=jnp.float32)
        m_ref[...] = m_new

    def _scores():
        q = q_ref[0, :, 0].reshape(_RQ, _D).astype(jnp.bfloat16)
        k = k_ref[0].astype(jnp.bfloat16)
        s = lax.dot_general(q, k, (((1,), (1,)), ((), ())),
                            preferred_element_type=jnp.float32)
        return s * (1.0 / math.sqrt(_D))

    @pl.when(kb < hi)
    def _full():
        _update(_scores())

    @pl.when(kb == hi)
    def _diag():
        s = _scores()
        r = lax.broadcasted_iota(jnp.int32, (_RQ, _BK2), 0)
        ir = qi * _BQ + jnp.bitwise_and(r, _BQ - 1)
        jc = kb * _BK2 + lax.broadcasted_iota(jnp.int32, (_RQ, _BK2), 1)
        _update(jnp.where(jc <= ir, s, _NEG))

    @pl.when(kb == _NK2 - 1)
    def _out():
        o_ref[0, :, 0] = (acc_ref[...] / l_ref[...]).reshape(1, 4, _BQ, _D)[0]


def _attention(qg, kh, vh):
    return pl.pallas_call(
        _attn_body,
        grid=(_KVH, _NQ, _NK2),
        in_specs=[
            pl.BlockSpec((1, _REP, 1, _BQ, _D),
                         lambda g, i, j: (g, 0, i, 0, 0)),
            pl.BlockSpec((1, _BK2, _D),
                         lambda g, i, j: (g, jnp.minimum(j, i // 2), 0)),
            pl.BlockSpec((1, _BK2, _D),
                         lambda g, i, j: (g, jnp.minimum(j, i // 2), 0)),
        ],
        out_specs=pl.BlockSpec((1, _REP, 1, _BQ, _D),
                               lambda g, i, j: (g, 0, i, 0, 0)),
        out_shape=jax.ShapeDtypeStruct((_KVH, _REP, _NQ, _BQ, _D),
                                       jnp.float32),
        scratch_shapes=[
            pltpu.VMEM((_RQ, 1), jnp.float32),
            pltpu.VMEM((_RQ, 1), jnp.float32),
            pltpu.VMEM((_RQ, _D), jnp.float32),
        ],
    )(qg, kh, vh)


def _post_body(a_ref, x_ref, wo_ref, w2_ref, wr_ref,
               x1_ref, h2_ref, probs_ref, wif_ref):
    a = a_ref[...].astype(jnp.bfloat16)
    x1 = x_ref[...] + jnp.dot(a, wo_ref[...],
                              preferred_element_type=jnp.float32)
    x1_ref[...] = x1
    nrm = jnp.mean(x1 * x1, axis=-1, keepdims=True)
    h2 = x1 * lax.rsqrt(nrm + _EPS) * w2_ref[...]
    h2_ref[...] = h2
    logits = jnp.dot(h2, wr_ref[...], preferred_element_type=jnp.float32)
    mx = jnp.max(logits, axis=-1, keepdims=True)
    ex = jnp.exp(logits - mx)
    probs = ex / jnp.sum(ex, axis=-1, keepdims=True)
    probs_ref[...] = probs
    io = lax.broadcasted_iota(jnp.int32, (_BT, _E), 1)
    m1 = jnp.max(probs, axis=-1, keepdims=True)
    i1 = jnp.min(jnp.where(probs == m1, io, _E), axis=-1, keepdims=True)
    masked = jnp.where(io == i1, -1.0, probs)
    m2 = jnp.max(masked, axis=-1, keepdims=True)
    i2 = jnp.min(jnp.where(masked == m2, io, _E), axis=-1, keepdims=True)
    tot = m1 + m2
    z = jnp.zeros((_BT, 1), jnp.float32)
    wif_ref[...] = jnp.concatenate(
        [m1 / tot, m2 / tot, i1.astype(jnp.float32), i2.astype(jnp.float32),
         z, z, z, z], axis=1)


def _post(a2d, x2d, wo, ln2_w, wr):
    n = _T // _BT
    return pl.pallas_call(
        _post_body,
        grid=(n,),
        in_specs=[
            pl.BlockSpec((_BT, _C), lambda i: (i, 0)),
            pl.BlockSpec((_BT, _C), lambda i: (i, 0)),
            pl.BlockSpec((_C, _C), lambda i: (0, 0)),
            pl.BlockSpec((1, _C), lambda i: (0, 0)),
            pl.BlockSpec((_C, _E), lambda i: (0, 0)),
        ],
        out_specs=[
            pl.BlockSpec((_BT, _C), lambda i: (i, 0)),
            pl.BlockSpec((_BT, _C), lambda i: (i, 0)),
            pl.BlockSpec((_BT, _E), lambda i: (i, 0)),
            pl.BlockSpec((_BT, _E), lambda i: (i, 0)),
        ],
        out_shape=[
            jax.ShapeDtypeStruct((_T, _C), jnp.float32),
            jax.ShapeDtypeStruct((_T, _C), jnp.float32),
            jax.ShapeDtypeStruct((_T, _E), jnp.float32),
            jax.ShapeDtypeStruct((_T, _E), jnp.float32),
        ],
    )(a2d, x2d, wo, ln2_w, wr)


def _route_body(wif_ref, probs_ref, tri_ref, d_ref, blk_ref, aux_ref):
    io = lax.broadcasted_iota(jnp.int32, (_T, _E), 1).astype(jnp.float32)
    e0 = wif_ref[:, 2:3]
    e1 = wif_ref[:, 3:4]
    oh0 = (io == e0).astype(jnp.float32)
    oh1 = (io == e1).astype(jnp.float32)
    ohs = oh0 + oh1
    excl = jnp.dot(tri_ref[...], ohs, preferred_element_type=jnp.float32)
    counts = jnp.sum(ohs, axis=0, keepdims=True)
    padded = jnp.floor((counts + (_BS - 1)) * (1.0 / _BS)) * _BS
    iou = lax.broadcasted_iota(jnp.int32, (_E, _E), 0)
    iol = lax.broadcasted_iota(jnp.int32, (_E, _E), 1)
    triu = (iou < iol).astype(jnp.float32)
    poff = jnp.dot(padded, triu, preferred_element_type=jnp.float32)
    r0 = jnp.sum(excl * oh0, axis=1, keepdims=True)
    r1 = jnp.sum(excl * oh1, axis=1, keepdims=True)
    d0 = jnp.sum(oh0 * poff, axis=1, keepdims=True) + r0
    d1 = jnp.sum(oh1 * poff, axis=1, keepdims=True) + r1
    z = jnp.zeros((_T, 1), jnp.float32)
    d_ref[...] = jnp.concatenate([d0, d1, z, z, z, z, z, z],
                                 axis=1).astype(jnp.int32)
    ioe = lax.broadcasted_iota(jnp.int32, (32, _E), 1).astype(jnp.float32)
    bs = (lax.broadcasted_iota(jnp.int32, (32, 1), 0) * _BS
          ).astype(jnp.float32)
    pend = poff + padded
    blk_e = jnp.sum((bs >= pend).astype(jnp.float32), axis=1, keepdims=True)
    ge = ((poff <= bs) & (padded > 0)).astype(jnp.float32)
    blk_e_use = jnp.clip(jnp.max((ioe + 1.0) * ge, axis=1, keepdims=True)
                         - 1.0, 0.0, _E - 1.0)
    sel = (ioe == blk_e_use).astype(jnp.float32)
    poff_use = jnp.sum(sel * poff, axis=1, keepdims=True)
    cnt_use = jnp.sum(sel * counts, axis=1, keepdims=True)
    valid = ((blk_e <= _E - 1.0) & (bs - poff_use < cnt_use))
    zb = jnp.zeros((32, 1), jnp.float32)
    blk_ref[...] = jnp.concatenate(
        [blk_e_use, valid.astype(jnp.float32), zb, zb, zb, zb, zb, zb],
        axis=1).astype(jnp.int32)
    pm = jnp.sum(probs_ref[...], axis=0, keepdims=True) * (1.0 / _T)
    f = counts * (1.0 / (_T * _K))
    aux_ref[...] = _E * jnp.sum(f * pm, axis=-1, keepdims=True)


def _route(wif, probs, tri):
    return pl.pallas_call(
        _route_body,
        out_shape=[
            jax.ShapeDtypeStruct((_T, _E), jnp.int32),
            jax.ShapeDtypeStruct((32, _E), jnp.int32),
            jax.ShapeDtypeStruct((1, 1), jnp.float32),
        ],
    )(wif, probs, tri)


def _ffn_body(se_ref, sv_ref, xs_ref, w1_ref, w2_ref, w3_ref, ys_ref):
    b = pl.program_id(0)

    @pl.when(sv_ref[b] == 1)
    def _go():
        x = xs_ref[...].astype(jnp.bfloat16)
        g = jnp.dot(x, w2_ref[0].astype(jnp.bfloat16),
                    preferred_element_type=jnp.float32)
        u = jnp.dot(x, w1_ref[0].astype(jnp.bfloat16),
                    preferred_element_type=jnp.float32)
        act = (g * jax.nn.sigmoid(g) * u).astype(jnp.bfloat16)
        ys_ref[...] = jnp.dot(act, w3_ref[0].astype(jnp.bfloat16),
                              preferred_element_type=jnp.float32)


def _ffn(xs, w1, w2, w3, blk_e, blk_v):
    grid_spec = pltpu.PrefetchScalarGridSpec(
        num_scalar_prefetch=2,
        grid=(_NB,),
        in_specs=[
            pl.BlockSpec((_BS, _C), lambda b, se, sv: (b, 0)),
            pl.BlockSpec((1, _C, _F), lambda b, se, sv: (se[b], 0, 0)),
            pl.BlockSpec((1, _C, _F), lambda b, se, sv: (se[b], 0, 0)),
            pl.BlockSpec((1, _F, _C), lambda b, se, sv: (se[b], 0, 0)),
        ],
        out_specs=pl.BlockSpec((_BS, _C), lambda b, se, sv: (b, 0)),
    )
    return pl.pallas_call(
        _ffn_body,
        grid_spec=grid_spec,
        out_shape=jax.ShapeDtypeStruct((_P, _C), jnp.float32),
    )(blk_e, blk_v, xs, w1, w2, w3)


def _combine_body(x1_ref, y0_ref, y1_ref, wif_ref, out_ref):
    w0 = wif_ref[:, 0:1]
    w1 = wif_ref[:, 1:2]
    out_ref[...] = x1_ref[...] + w0 * y0_ref[...] + w1 * y1_ref[...]


def _combine(x1, yg, wif):
    n = _T // _BT
    return pl.pallas_call(
        _combine_body,
        grid=(n,),
        in_specs=[
            pl.BlockSpec((_BT, _C), lambda i: (i, 0)),
            pl.BlockSpec((_BT, _C), lambda i: (i, 0)),
            pl.BlockSpec((_BT, _C), lambda i: (i + n, 0)),
            pl.BlockSpec((_BT, _E), lambda i: (i, 0)),
        ],
        out_specs=pl.BlockSpec((_BT, _C), lambda i: (i, 0)),
        out_shape=jax.ShapeDtypeStruct((_T, _C), jnp.float32),
    )(x1, yg, yg, wif)


def _gather_rows(table, idx):
    rows, width = idx.shape[0], table.shape[1]
    info = plsc.get_sparse_core_info()
    nw = info.num_cores * info.num_subcores
    b_per_w = rows // nw
    chunk = 32
    nchunks = b_per_w // chunk
    nbuf = min(3, nchunks)
    mesh = plsc.VectorSubcoreMesh(core_axis_name="c", subcore_axis_name="s")

    @functools.partial(
        pl.kernel, mesh=mesh,
        out_type=jax.ShapeDtypeStruct((rows, width), jnp.float32),
        scratch_types=(
            [pltpu.VMEM((chunk,), jnp.int32) for _ in range(nbuf)]
            + [pltpu.VMEM((chunk, width), jnp.float32) for _ in range(nbuf)]
            + [pltpu.SemaphoreType.DMA for _ in range(2 * nbuf)]
        ),
    )
    def k(table_hbm, idx_hbm, out_hbm, *scr):
        idx_v = scr[:nbuf]
        rows_v = scr[nbuf:2 * nbuf]
        gsem = scr[2 * nbuf:3 * nbuf]
        osem = scr[3 * nbuf:4 * nbuf]
        wid = lax.axis_index("s") * info.num_cores + lax.axis_index("c")
        base = wid * b_per_w
        ghandle = [None] * nbuf
        ohandle = [None] * nbuf

        for ci in range(nchunks):
            b = ci % nbuf
            if ci >= nbuf:
                ohandle[b].wait()
            pltpu.sync_copy(idx_hbm.at[pl.ds(base + ci * chunk, chunk)],
                            idx_v[b])
            ghandle[b] = pltpu.async_copy(table_hbm.at[idx_v[b]],
                                          rows_v[b], gsem[b])
            if ci >= 1:
                p = (ci - 1) % nbuf
                ghandle[p].wait()
                ohandle[p] = pltpu.async_copy(
                    rows_v[p],
                    out_hbm.at[pl.ds(base + (ci - 1) * chunk, chunk)],
                    osem[p])
        last = nchunks - 1
        ghandle[last % nbuf].wait()
        ohandle[last % nbuf] = pltpu.async_copy(
            rows_v[last % nbuf],
            out_hbm.at[pl.ds(base + last * chunk, chunk)],
            osem[last % nbuf])
        for ci in range(max(0, nchunks - nbuf), nchunks):
            ohandle[ci % nbuf].wait()

    return k(table, idx)


def _scatter_rows(src, d0, d1):
    t_rows, width = src.shape
    info = plsc.get_sparse_core_info()
    nw = info.num_cores * info.num_subcores
    b_per_w = t_rows // nw
    chunk = 32
    nchunks = b_per_w // chunk
    nbuf = min(2, nchunks)
    mesh = plsc.VectorSubcoreMesh(core_axis_name="c", subcore_axis_name="s")

    @functools.partial(
        pl.kernel, mesh=mesh,
        out_type=jax.ShapeDtypeStruct((_P, width), jnp.float32),
        scratch_types=(
            [pltpu.VMEM((chunk,), jnp.int32) for _ in range(2 * nbuf)]
            + [pltpu.VMEM((chunk, width), jnp.float32) for _ in range(nbuf)]
            + [pltpu.SemaphoreType.DMA for _ in range(2 * nbuf)]
        ),
    )
    def k(src_hbm, d0_hbm, d1_hbm, out_hbm, *scr):
        i0 = scr[:nbuf]
        i1 = scr[nbuf:2 * nbuf]
        rows_v = scr[2 * nbuf:3 * nbuf]
        s0 = scr[3 * nbuf:4 * nbuf]
        s1 = scr[4 * nbuf:5 * nbuf]
        wid = lax.axis_index("s") * info.num_cores + lax.axis_index("c")
        base = wid * b_per_w
        h0 = [None] * nbuf
        h1 = [None] * nbuf
        for ci in range(nchunks):
            b = ci % nbuf
            if ci >= nbuf:
                h0[b].wait()
                h1[b].wait()
            off = base + ci * chunk
            pltpu.sync_copy(src_hbm.at[pl.ds(off, chunk)], rows_v[b])
            pltpu.sync_copy(d0_hbm.at[pl.ds(off, chunk)], i0[b])
            pltpu.sync_copy(d1_hbm.at[pl.ds(off, chunk)], i1[b])
            h0[b] = pltpu.async_copy(rows_v[b], out_hbm.at[i0[b]], s0[b])
            h1[b] = pltpu.async_copy(rows_v[b], out_hbm.at[i1[b]], s1[b])
        for b in range(nbuf):
            if h0[b] is not None:
                h0[b].wait()
                h1[b].wait()

    return k(src, d0, d1)


def _rope_tables(cos, sin):
    cq = jnp.tile(cos, (1, _H))
    sq = jnp.tile(sin, (1, _H))
    ck = jnp.tile(cos, (1, _KVH))
    sk = jnp.tile(sin, (1, _KVH))
    return cq, sq, ck, sk


def _perm_cols(w, heads):
    wr = w.reshape(_C, heads, _D // 2, 2)
    we = wr[:, :, :, 0].reshape(_C, heads * _D // 2)
    wo = wr[:, :, :, 1].reshape(_C, heads * _D // 2)
    return jnp.concatenate([we, wo], axis=1)


def kernel(x, cos, sin, ln1_w, Wq, Wk, Wv, Wo, ln2_w, Wr, W1, W2, W3):
    x2d = x.reshape(_T, _C)
    cq, sq, ck, sk = _rope_tables(cos, sin)
    wq_p = _perm_cols(Wq, _H).astype(jnp.bfloat16)
    wk_p = _perm_cols(Wk, _KVH).astype(jnp.bfloat16)

    q, k, v = _proj(x2d, ln1_w.reshape(1, _C), wq_p, wk_p,
                    Wv.astype(jnp.bfloat16), cq, sq, ck, sk)

    hq = _H * _D // 2
    hk = _KVH * _D // 2
    qh = jnp.concatenate(
        [q[:, :hq].reshape(_T, _H, _D // 2),
         q[:, hq:].reshape(_T, _H, _D // 2)], axis=-1).transpose(1, 0, 2)
    kh = jnp.concatenate(
        [k[:, :hk].reshape(_T, _KVH, _D // 2),
         k[:, hk:].reshape(_T, _KVH, _D // 2)], axis=-1).transpose(1, 0, 2)
    vh = v.reshape(_T, _KVH, _D).transpose(1, 0, 2)

    qg = qh.reshape(_KVH, _REP, _NQ, _BQ, _D)
    ah = _attention(qg, kh, vh).reshape(_H, _T, _D)
    a2d = ah.transpose(1, 0, 2).reshape(_T, _C)

    x1, h2, probs, wif = _post(a2d, x2d, Wo.astype(jnp.bfloat16),
                               ln2_w.reshape(1, _C), Wr)

    tri = jnp.tril(jnp.ones((_T, _T), jnp.float32), -1)
    darr, blk, auxm = _route(wif, probs, tri)
    aux = auxm[0, 0]
    d0 = darr[:, 0]
    d1 = darr[:, 1]
    idx_flat = jnp.concatenate([d0, d1])
    blk_e_use = blk[:_NB, 0]
    blk_v = blk[:_NB, 1]

    xs = _scatter_rows(h2, d0, d1)
    ys = _ffn(xs, W1, W2, W3, blk_e_use, blk_v)
    yg = _gather_rows(ys, idx_flat)
    out = _combine(x1, yg, wif)

    return out.reshape(_B, _T, _C), aux


# speedup vs baseline: 1.0994x; 1.0994x over previous
import functools
import math

import jax
import jax.numpy as jnp
from jax import lax
from jax.experimental import pallas as pl
from jax.experimental.pallas import tpu as pltpu
from jax.experimental.pallas import tpu_sc as plsc

_B, _T, _C = 1, 2048, 1024
_H, _KVH, _D = 16, 4, 64
_E, _K, _F = 8, 2, 2048
_EPS = 1e-5
_BT = 256
_BQ, _BK = 256, 256
_NQ, _NK = _T // _BQ, _T // _BK
_BS = 256
_NB = (_K * _T) // _BS + _E
_P = _NB * _BS
_NEG = -1e30


def _proj_body(x_ref, w_ref, wq_ref, wk_ref, wv_ref, cq_ref, sq_ref,
               ck_ref, sk_ref, q_ref, k_ref, v_ref):
    x = x_ref[...]
    nrm = jnp.mean(x * x, axis=-1, keepdims=True)
    h = (x * lax.rsqrt(nrm + _EPS) * w_ref[...]).astype(jnp.bfloat16)
    q = jnp.dot(h, wq_ref[...], preferred_element_type=jnp.float32)
    k = jnp.dot(h, wk_ref[...], preferred_element_type=jnp.float32)
    v = jnp.dot(h, wv_ref[...], preferred_element_type=jnp.float32)
    hq = _H * _D // 2
    hk = _KVH * _D // 2
    qe, qo = q[:, :hq], q[:, hq:]
    ke, ko = k[:, :hk], k[:, hk:]
    cq, sq = cq_ref[...], sq_ref[...]
    ck, sk = ck_ref[...], sk_ref[...]
    y1q = qe * cq - qo * sq
    y2q = qe * sq + qo * cq
    y1k = ke * ck - ko * sk
    y2k = ke * sk + ko * ck
    hw = _D // 2
    for h in range(_H):
        q_ref[h // _REP, h % _REP, 0] = jnp.concatenate(
            [y1q[:, h * hw:(h + 1) * hw], y2q[:, h * hw:(h + 1) * hw]],
            axis=1)
    for g in range(_KVH):
        k_ref[g] = jnp.concatenate(
            [y1k[:, g * hw:(g + 1) * hw], y2k[:, g * hw:(g + 1) * hw]],
            axis=1)
        v_ref[g] = v[:, g * _D:(g + 1) * _D]


def _proj(x2d, ln1_w, wq_p, wk_p, wv, cq, sq, ck, sk):
    n = _T // _BT
    return pl.pallas_call(
        _proj_body,
        grid=(n,),
        in_specs=[
            pl.BlockSpec((_BT, _C), lambda i: (i, 0)),
            pl.BlockSpec((1, _C), lambda i: (0, 0)),
            pl.BlockSpec((_C, _H * _D), lambda i: (0, 0)),
            pl.BlockSpec((_C, _KVH * _D), lambda i: (0, 0)),
            pl.BlockSpec((_C, _KVH * _D), lambda i: (0, 0)),
            pl.BlockSpec((_BT, _H * _D // 2), lambda i: (i, 0)),
            pl.BlockSpec((_BT, _H * _D // 2), lambda i: (i, 0)),
            pl.BlockSpec((_BT, _KVH * _D // 2), lambda i: (i, 0)),
            pl.BlockSpec((_BT, _KVH * _D // 2), lambda i: (i, 0)),
        ],
        out_specs=[
            pl.BlockSpec((_KVH, _H // _KVH, 1, _BT, _D),
                         lambda i: (0, 0, i, 0, 0)),
            pl.BlockSpec((_KVH, _BT, _D), lambda i: (0, i, 0)),
            pl.BlockSpec((_KVH, _BT, _D), lambda i: (0, i, 0)),
        ],
        out_shape=[
            jax.ShapeDtypeStruct((_KVH, _H // _KVH, _T // _BT, _BT, _D),
                                 jnp.float32),
            jax.ShapeDtypeStruct((_KVH, _T, _D), jnp.float32),
            jax.ShapeDtypeStruct((_KVH, _T, _D), jnp.float32),
        ],
    )(x2d, ln1_w, wq_p, wk_p, wv, cq, sq, ck, sk)


_REP = _H // _KVH
_RQ = _REP * _BQ
_BK2 = 512
_NK2 = _T // _BK2


def _attn_body(q_ref, k_ref, v_ref, o_ref, m_ref, l_ref, acc_ref):
    qi = pl.program_id(1)
    kb = pl.program_id(2)
    hi = qi // (_BK2 // _BQ)

    @pl.when(kb == 0)
    def _init():
        m_ref[...] = jnp.full_like(m_ref, _NEG)
        l_ref[...] = jnp.zeros_like(l_ref)
        acc_ref[...] = jnp.zeros_like(acc_ref)

    def _update(s):
        v = v_ref[0].astype(jnp.bfloat16)
        m_prev = m_ref[...]
        m_new = jnp.maximum(m_prev, jnp.max(s, axis=1, keepdims=True))
        alpha = jnp.exp(m_prev - m_new)
        p = jnp.exp(s - m_new)
        l_ref[...] = l_ref[...] * alpha + jnp.sum(p, axis=1, keepdims=True)
        acc_ref[...] = acc_ref[...] * alpha + jnp.dot(
            p.astype(jnp.bfloat16), v, preferred_element_type=jnp.float32)
        m_ref[...] = m_new

    def _scores():
        q = q_ref[0, :, 0].reshape(_RQ, _D).astype(jnp.bfloat16)
        k = k_ref[0].astype(jnp.bfloat16)
        s = lax.dot_general(q, k, (((1,), (1,)), ((), ())),
                            preferred_element_type=jnp.float32)
        return s * (1.0 / math.sqrt(_D))

    @pl.when(kb < hi)
    def _full():
        _update(_scores())

    @pl.when(kb == hi)
    def _diag():
        s = _scores()
        r = lax.broadcasted_iota(jnp.int32, (_RQ, _BK2), 0)
        ir = qi * _BQ + jnp.bitwise_and(r, _BQ - 1)
        jc = kb * _BK2 + lax.broadcasted_iota(jnp.int32, (_RQ, _BK2), 1)
        _update(jnp.where(jc <= ir, s, _NEG))

    @pl.when(kb == _NK2 - 1)
    def _out():
        a = acc_ref[...] / l_ref[...]
        for r in range(_REP):
            o_ref[:, r * _D:(r + 1) * _D] = a[r * _BQ:(r + 1) * _BQ, :]


def _attention(qg, kh, vh):
    return pl.pallas_call(
        _attn_body,
        grid=(_KVH, _NQ, _NK2),
        in_specs=[
            pl.BlockSpec((1, _REP, 1, _BQ, _D),
                         lambda g, i, j: (g, 0, i, 0, 0)),
            pl.BlockSpec((1, _BK2, _D),
                         lambda g, i, j: (g, jnp.minimum(j, i // 2), 0)),
            pl.BlockSpec((1, _BK2, _D),
                         lambda g, i, j: (g, jnp.minimum(j, i // 2), 0)),
        ],
        out_specs=pl.BlockSpec((_BQ, _REP * _D), lambda g, i, j: (i, g)),
        out_shape=jax.ShapeDtypeStruct((_T, _C), jnp.float32),
        scratch_shapes=[
            pltpu.VMEM((_RQ, 1), jnp.float32),
            pltpu.VMEM((_RQ, 1), jnp.float32),
            pltpu.VMEM((_RQ, _D), jnp.float32),
        ],
    )(qg, kh, vh)


def _post_body(a_ref, x_ref, wo_ref, w2_ref, wr_ref,
               x1_ref, h2_ref, probs_ref, wif_ref):
    a = a_ref[...].astype(jnp.bfloat16)
    x1 = x_ref[...] + jnp.dot(a, wo_ref[...],
                              preferred_element_type=jnp.float32)
    x1_ref[...] = x1
    nrm = jnp.mean(x1 * x1, axis=-1, keepdims=True)
    h2 = x1 * lax.rsqrt(nrm + _EPS) * w2_ref[...]
    h2_ref[...] = h2
    logits = jnp.dot(h2, wr_ref[...], preferred_element_type=jnp.float32)
    mx = jnp.max(logits, axis=-1, keepdims=True)
    ex = jnp.exp(logits - mx)
    probs = ex / jnp.sum(ex, axis=-1, keepdims=True)
    probs_ref[...] = probs
    io = lax.broadcasted_iota(jnp.int32, (_BT, _E), 1)
    m1 = jnp.max(probs, axis=-1, keepdims=True)
    i1 = jnp.min(jnp.where(probs == m1, io, _E), axis=-1, keepdims=True)
    masked = jnp.where(io == i1, -1.0, probs)
    m2 = jnp.max(masked, axis=-1, keepdims=True)
    i2 = jnp.min(jnp.where(masked == m2, io, _E), axis=-1, keepdims=True)
    tot = m1 + m2
    z = jnp.zeros((_BT, 1), jnp.float32)
    wif_ref[...] = jnp.concatenate(
        [m1 / tot, m2 / tot, i1.astype(jnp.float32), i2.astype(jnp.float32),
         z, z, z, z], axis=1)


def _post(a2d, x2d, wo, ln2_w, wr):
    n = _T // _BT
    return pl.pallas_call(
        _post_body,
        grid=(n,),
        in_specs=[
            pl.BlockSpec((_BT, _C), lambda i: (i, 0)),
            pl.BlockSpec((_BT, _C), lambda i: (i, 0)),
            pl.BlockSpec((_C, _C), lambda i: (0, 0)),
            pl.BlockSpec((1, _C), lambda i: (0, 0)),
            pl.BlockSpec((_C, _E), lambda i: (0, 0)),
        ],
        out_specs=[
            pl.BlockSpec((_BT, _C), lambda i: (i, 0)),
            pl.BlockSpec((_BT, _C), lambda i: (i, 0)),
            pl.BlockSpec((_BT, _E), lambda i: (i, 0)),
            pl.BlockSpec((_BT, _E), lambda i: (i, 0)),
        ],
        out_shape=[
            jax.ShapeDtypeStruct((_T, _C), jnp.float32),
            jax.ShapeDtypeStruct((_T, _C), jnp.float32),
            jax.ShapeDtypeStruct((_T, _E), jnp.float32),
            jax.ShapeDtypeStruct((_T, _E), jnp.float32),
        ],
    )(a2d, x2d, wo, ln2_w, wr)


def _route_body(wif_ref, probs_ref, tri_ref, d_ref, blk_ref, aux_ref):
    io = lax.broadcasted_iota(jnp.int32, (_T, _E), 1).astype(jnp.float32)
    e0 = wif_ref[:, 2:3]
    e1 = wif_ref[:, 3:4]
    oh0 = (io == e0).astype(jnp.float32)
    oh1 = (io == e1).astype(jnp.float32)
    ohs = oh0 + oh1
    excl = jnp.dot(tri_ref[...], ohs, preferred_element_type=jnp.float32)
    counts = jnp.sum(ohs, axis=0, keepdims=True)
    padded = jnp.floor((counts + (_BS - 1)) * (1.0 / _BS)) * _BS
    iou = lax.broadcasted_iota(jnp.int32, (_E, _E), 0)
    iol = lax.broadcasted_iota(jnp.int32, (_E, _E), 1)
    triu = (iou < iol).astype(jnp.float32)
    poff = jnp.dot(padded, triu, preferred_element_type=jnp.float32)
    r0 = jnp.sum(excl * oh0, axis=1, keepdims=True)
    r1 = jnp.sum(excl * oh1, axis=1, keepdims=True)
    d0 = jnp.sum(oh0 * poff, axis=1, keepdims=True) + r0
    d1 = jnp.sum(oh1 * poff, axis=1, keepdims=True) + r1
    z = jnp.zeros((_T, 1), jnp.float32)
    d_ref[...] = jnp.concatenate([d0, d1, z, z, z, z, z, z],
                                 axis=1).astype(jnp.int32)
    ioe = lax.broadcasted_iota(jnp.int32, (32, _E), 1).astype(jnp.float32)
    bs = (lax.broadcasted_iota(jnp.int32, (32, 1), 0) * _BS
          ).astype(jnp.float32)
    pend = poff + padded
    blk_e = jnp.sum((bs >= pend).astype(jnp.float32), axis=1, keepdims=True)
    ge = ((poff <= bs) & (padded > 0)).astype(jnp.float32)
    blk_e_use = jnp.clip(jnp.max((ioe + 1.0) * ge, axis=1, keepdims=True)
                         - 1.0, 0.0, _E - 1.0)
    sel = (ioe == blk_e_use).astype(jnp.float32)
    poff_use = jnp.sum(sel * poff, axis=1, keepdims=True)
    cnt_use = jnp.sum(sel * counts, axis=1, keepdims=True)
    valid = ((blk_e <= _E - 1.0) & (bs - poff_use < cnt_use))
    zb = jnp.zeros((32, 1), jnp.float32)
    blk_ref[...] = jnp.concatenate(
        [blk_e_use, valid.astype(jnp.float32), zb, zb, zb, zb, zb, zb],
        axis=1).astype(jnp.int32)
    pm = jnp.sum(probs_ref[...], axis=0, keepdims=True) * (1.0 / _T)
    f = counts * (1.0 / (_T * _K))
    aux_ref[...] = _E * jnp.sum(f * pm, axis=-1, keepdims=True)


def _route(wif, probs, tri):
    return pl.pallas_call(
        _route_body,
        out_shape=[
            jax.ShapeDtypeStruct((_T, _E), jnp.int32),
            jax.ShapeDtypeStruct((32, _E), jnp.int32),
            jax.ShapeDtypeStruct((1, 1), jnp.float32),
        ],
    )(wif, probs, tri)


def _ffn_body(se_ref, sv_ref, xs_ref, w1_ref, w2_ref, w3_ref, ys_ref):
    b = pl.program_id(0)

    @pl.when(sv_ref[b] == 1)
    def _go():
        x = xs_ref[...].astype(jnp.bfloat16)
        g = jnp.dot(x, w2_ref[0].astype(jnp.bfloat16),
                    preferred_element_type=jnp.float32)
        u = jnp.dot(x, w1_ref[0].astype(jnp.bfloat16),
                    preferred_element_type=jnp.float32)
        act = (g * jax.nn.sigmoid(g) * u).astype(jnp.bfloat16)
        ys_ref[...] = jnp.dot(act, w3_ref[0].astype(jnp.bfloat16),
                              preferred_element_type=jnp.float32)


def _ffn(xs, w1, w2, w3, blk_e, blk_v):
    grid_spec = pltpu.PrefetchScalarGridSpec(
        num_scalar_prefetch=2,
        grid=(_NB,),
        in_specs=[
            pl.BlockSpec((_BS, _C), lambda b, se, sv: (b, 0)),
            pl.BlockSpec((1, _C, _F), lambda b, se, sv: (se[b], 0, 0)),
            pl.BlockSpec((1, _C, _F), lambda b, se, sv: (se[b], 0, 0)),
            pl.BlockSpec((1, _F, _C), lambda b, se, sv: (se[b], 0, 0)),
        ],
        out_specs=pl.BlockSpec((_BS, _C), lambda b, se, sv: (b, 0)),
    )
    return pl.pallas_call(
        _ffn_body,
        grid_spec=grid_spec,
        out_shape=jax.ShapeDtypeStruct((_P, _C), jnp.float32),
    )(blk_e, blk_v, xs, w1, w2, w3)


def _combine_body(x1_ref, y0_ref, y1_ref, wif_ref, out_ref):
    w0 = wif_ref[:, 0:1]
    w1 = wif_ref[:, 1:2]
    out_ref[...] = x1_ref[...] + w0 * y0_ref[...] + w1 * y1_ref[...]


def _combine(x1, yg, wif):
    n = _T // _BT
    return pl.pallas_call(
        _combine_body,
        grid=(n,),
        in_specs=[
            pl.BlockSpec((_BT, _C), lambda i: (i, 0)),
            pl.BlockSpec((_BT, _C), lambda i: (i, 0)),
            pl.BlockSpec((_BT, _C), lambda i: (i + n, 0)),
            pl.BlockSpec((_BT, _E), lambda i: (i, 0)),
        ],
        out_specs=pl.BlockSpec((_BT, _C), lambda i: (i, 0)),
        out_shape=jax.ShapeDtypeStruct((_T, _C), jnp.float32),
    )(x1, yg, yg, wif)


def _gather_rows(table, idx):
    rows, width = idx.shape[0], table.shape[1]
    info = plsc.get_sparse_core_info()
    nw = info.num_cores * info.num_subcores
    b_per_w = rows // nw
    chunk = 32
    nchunks = b_per_w // chunk
    nbuf = min(3, nchunks)
    mesh = plsc.VectorSubcoreMesh(core_axis_name="c", subcore_axis_name="s")

    @functools.partial(
        pl.kernel, mesh=mesh,
        out_type=jax.ShapeDtypeStruct((rows, width), jnp.float32),
        scratch_types=(
            [pltpu.VMEM((chunk,), jnp.int32) for _ in range(nbuf)]
            + [pltpu.VMEM((chunk, width), jnp.float32) for _ in range(nbuf)]
            + [pltpu.SemaphoreType.DMA for _ in range(2 * nbuf)]
        ),
    )
    def k(table_hbm, idx_hbm, out_hbm, *scr):
        idx_v = scr[:nbuf]
        rows_v = scr[nbuf:2 * nbuf]
        gsem = scr[2 * nbuf:3 * nbuf]
        osem = scr[3 * nbuf:4 * nbuf]
        wid = lax.axis_index("s") * info.num_cores + lax.axis_index("c")
        base = wid * b_per_w
        ghandle = [None] * nbuf
        ohandle = [None] * nbuf

        for ci in range(nchunks):
            b = ci % nbuf
            if ci >= nbuf:
                ohandle[b].wait()
            pltpu.sync_copy(idx_hbm.at[pl.ds(base + ci * chunk, chunk)],
                            idx_v[b])
            ghandle[b] = pltpu.async_copy(table_hbm.at[idx_v[b]],
                                          rows_v[b], gsem[b])
            if ci >= 1:
                p = (ci - 1) % nbuf
                ghandle[p].wait()
                ohandle[p] = pltpu.async_copy(
                    rows_v[p],
                    out_hbm.at[pl.ds(base + (ci - 1) * chunk, chunk)],
                    osem[p])
        last = nchunks - 1
        ghandle[last % nbuf].wait()
        ohandle[last % nbuf] = pltpu.async_copy(
            rows_v[last % nbuf],
            out_hbm.at[pl.ds(base + last * chunk, chunk)],
            osem[last % nbuf])
        for ci in range(max(0, nchunks - nbuf), nchunks):
            ohandle[ci % nbuf].wait()

    return k(table, idx)


def _scatter_rows(src, d0, d1):
    t_rows, width = src.shape
    info = plsc.get_sparse_core_info()
    nw = info.num_cores * info.num_subcores
    b_per_w = t_rows // nw
    chunk = 32
    nchunks = b_per_w // chunk
    nbuf = min(2, nchunks)
    mesh = plsc.VectorSubcoreMesh(core_axis_name="c", subcore_axis_name="s")

    @functools.partial(
        pl.kernel, mesh=mesh,
        out_type=jax.ShapeDtypeStruct((_P, width), jnp.float32),
        scratch_types=(
            [pltpu.VMEM((chunk,), jnp.int32) for _ in range(2 * nbuf)]
            + [pltpu.VMEM((chunk, width), jnp.float32) for _ in range(nbuf)]
            + [pltpu.SemaphoreType.DMA for _ in range(2 * nbuf)]
        ),
    )
    def k(src_hbm, d0_hbm, d1_hbm, out_hbm, *scr):
        i0 = scr[:nbuf]
        i1 = scr[nbuf:2 * nbuf]
        rows_v = scr[2 * nbuf:3 * nbuf]
        s0 = scr[3 * nbuf:4 * nbuf]
        s1 = scr[4 * nbuf:5 * nbuf]
        wid = lax.axis_index("s") * info.num_cores + lax.axis_index("c")
        base = wid * b_per_w
        h0 = [None] * nbuf
        h1 = [None] * nbuf
        for ci in range(nchunks):
            b = ci % nbuf
            if ci >= nbuf:
                h0[b].wait()
                h1[b].wait()
            off = base + ci * chunk
            pltpu.sync_copy(src_hbm.at[pl.ds(off, chunk)], rows_v[b])
            pltpu.sync_copy(d0_hbm.at[pl.ds(off, chunk)], i0[b])
            pltpu.sync_copy(d1_hbm.at[pl.ds(off, chunk)], i1[b])
            h0[b] = pltpu.async_copy(rows_v[b], out_hbm.at[i0[b]], s0[b])
            h1[b] = pltpu.async_copy(rows_v[b], out_hbm.at[i1[b]], s1[b])
        for b in range(nbuf):
            if h0[b] is not None:
                h0[b].wait()
                h1[b].wait()

    return k(src, d0, d1)


def _rope_tables(cos, sin):
    cq = jnp.tile(cos, (1, _H))
    sq = jnp.tile(sin, (1, _H))
    ck = jnp.tile(cos, (1, _KVH))
    sk = jnp.tile(sin, (1, _KVH))
    return cq, sq, ck, sk


def _perm_cols(w, heads):
    wr = w.reshape(_C, heads, _D // 2, 2)
    we = wr[:, :, :, 0].reshape(_C, heads * _D // 2)
    wo = wr[:, :, :, 1].reshape(_C, heads * _D // 2)
    return jnp.concatenate([we, wo], axis=1)


def kernel(x, cos, sin, ln1_w, Wq, Wk, Wv, Wo, ln2_w, Wr, W1, W2, W3):
    x2d = x.reshape(_T, _C)
    cq, sq, ck, sk = _rope_tables(cos, sin)
    wq_p = _perm_cols(Wq, _H).astype(jnp.bfloat16)
    wk_p = _perm_cols(Wk, _KVH).astype(jnp.bfloat16)

    qg, kh, vh = _proj(x2d, ln1_w.reshape(1, _C), wq_p, wk_p,
                       Wv.astype(jnp.bfloat16), cq, sq, ck, sk)
    a2d = _attention(qg, kh, vh)

    x1, h2, probs, wif = _post(a2d, x2d, Wo.astype(jnp.bfloat16),
                               ln2_w.reshape(1, _C), Wr)

    tri = jnp.tril(jnp.ones((_T, _T), jnp.float32), -1)
    darr, blk, auxm = _route(wif, probs, tri)
    aux = auxm[0, 0]
    d0 = darr[:, 0]
    d1 = darr[:, 1]
    idx_flat = jnp.concatenate([d0, d1])
    blk_e_use = blk[:_NB, 0]
    blk_v = blk[:_NB, 1]

    xs = _scatter_rows(h2, d0, d1)
    ys = _ffn(xs, W1, W2, W3, blk_e_use, blk_v)
    yg = _gather_rows(ys, idx_flat)
    out = _combine(x1, yg, wif)

    return out.reshape(_B, _T, _C), aux
